# baseline (speedup 1.0000x reference)
import jax
import jax.numpy as jnp
from jax import lax
from jax.experimental import pallas as pl
from jax.experimental.pallas import tpu as pltpu
from jax.experimental.pallas import tpu_sc as plsc

N_NODES = 1000
EP = 12288
NW = 32
EPW = EP // NW
NPAD = 1024
LPL = EP // 16

_PI = 3.141592653589793
_HALF_PI = 1.5707963267948966


def _atan_poly(a):
    z = a * a
    p = jnp.float32(-0.0117212)
    p = p * z + jnp.float32(0.05265332)
    p = p * z + jnp.float32(-0.11643287)
    p = p * z + jnp.float32(0.19354346)
    p = p * z + jnp.float32(-0.33262347)
    p = p * z + jnp.float32(0.99997726)
    return a * p


def _atan2_pos(y, x):
    ax = jnp.abs(x)
    mn = jnp.minimum(y, ax)
    mx = jnp.maximum(y, ax)
    a = jnp.where(mx > 0.0, mn / mx, 0.0)
    r = _atan_poly(a)
    r = jnp.where(y > ax, _HALF_PI - r, r)
    r = jnp.where(x < 0.0, _PI - r, r)
    return r


def _sqrt_nn(x):
    i = plsc.bitcast(x, jnp.int32)
    i = jnp.int32(0x5F3759DF) - lax.shift_right_logical(i, 1)
    y = plsc.bitcast(i, jnp.float32)
    for _ in range(3):
        y = y * (jnp.float32(1.5) - jnp.float32(0.5) * x * y * y)
    return jnp.where(x > 0.0, x * y, 0.0)


def _sc_body(recv_hbm, send_hbm, ex_hbm, ey_hbm, ez_hbm, vx_hbm, vy_hbm, vz_hbm,
             azk_hbm, azb_hbm, ck_hbm,
             az_hbm,
             recv_s, ex_s, ey_s, ez_s, sidx_s, cnt_s, tot_s, off_s,
             send_s, vx_s, vy_s, vz_s, t_s, p_s, d_s, az_s,
             azk_s, azb_s, ck_s):
    cid = lax.axis_index("c")
    sid = lax.axis_index("s")
    wid = sid * 2 + cid
    base = wid * EPW

    lanes = jnp.arange(16, dtype=jnp.int32)
    ones_i = jnp.ones((16,), jnp.int32)

    pltpu.sync_copy(azk_hbm, azk_s)
    pltpu.sync_copy(azb_hbm, azb_s)
    pltpu.sync_copy(ck_hbm, ck_s)

    pltpu.sync_copy(recv_hbm, recv_s)
    pltpu.sync_copy(ex_hbm, ex_s)
    pltpu.sync_copy(ey_hbm, ey_s)
    pltpu.sync_copy(ez_hbm, ez_s)
    pltpu.sync_copy(send_hbm.at[pl.ds(base, EPW)], send_s)
    pltpu.sync_copy(vx_hbm.at[pl.ds(base, EPW)], vx_s)
    pltpu.sync_copy(vy_hbm.at[pl.ds(base, EPW)], vy_s)
    pltpu.sync_copy(vz_hbm.at[pl.ds(base, EPW)], vz_s)

    def wgt_body(m, carry):
        wph, wth, ba = carry
        ak0 = azk_s[pl.ds(m * 16, 16)]
        ak1 = azk_s[pl.ds(64 + m * 16, 16)]
        ab = azb_s[pl.ds(m * 16, 16)]
        for u in range(16):
            wa_k = ck_s[pl.ds((64 + m * 16 + u) * 16, 16)]
            wph = wph + ak0[u] * wa_k
            wth = wth + ak1[u] * wa_k
            ba = ba + ab[u] * wa_k
        return (wph, wth, ba)
    zf = jnp.zeros((16,), jnp.float32)
    wph_v, wth_v, ba_v = lax.fori_loop(0, 4, wgt_body, (zf, zf, zf))

    def zero_body(i, _):
        z = jnp.zeros((16,), jnp.int32)
        for u in range(4):
            cnt_s[pl.ds(i * 64 + u * 16, 16)] = z
        return 0
    lax.fori_loop(0, (16 * NPAD) // 64, zero_body, 0)

    def count_body(k, _):
        for u in range(2):
            eidx = lanes * LPL + (k * 2 + u)
            rv = plsc.load_gather(recv_s, [eidx])
            plsc.addupdate_scatter(cnt_s, [lanes * NPAD + rv], ones_i)
        return 0
    lax.fori_loop(0, LPL // 2, count_body, 0)

    def pfx_outer(m, _):
        run = jnp.zeros((16,), jnp.int32)
        for l in range(16):
            o = l * NPAD + m * 16
            cl = cnt_s[pl.ds(o, 16)]
            cnt_s[pl.ds(o, 16)] = run
            run = run + cl
        tot_s[pl.ds(m * 16, 16)] = run
        return 0
    lax.fori_loop(0, NPAD // 16, pfx_outer, 0)

    def off_body(m, carry):
        t = tot_s[pl.ds(m * 16, 16)]
        cs = plsc.cumsum(t)
        off_s[pl.ds(m * 16, 16)] = cs - t + carry
        return carry + jnp.sum(t)
    lax.fori_loop(0, NPAD // 16, off_body, jnp.int32(0))

    def place_body(k, _):
        for u in range(2):
            eidx = lanes * LPL + (k * 2 + u)
            rv = plsc.load_gather(recv_s, [eidx])
            cidx = lanes * NPAD + rv
            pos = plsc.load_gather(off_s, [rv]) + plsc.load_gather(cnt_s, [cidx])
            plsc.store_scatter(sidx_s, [pos], eidx)
            plsc.addupdate_scatter(cnt_s, [cidx], ones_i)
        return 0
    lax.fori_loop(0, LPL // 2, place_body, 0)

    def group_body(g, _):
        gbase = base + g * 16
        sv = send_s[pl.ds(g * 16, 16)]
        pxv = ex_s[pl.ds(gbase, 16)]
        pyv = ey_s[pl.ds(gbase, 16)]
        pzv = ez_s[pl.ds(gbase, 16)]
        vxv = vx_s[pl.ds(g * 16, 16)]
        vyv = vy_s[pl.ds(g * 16, 16)]
        vzv = vz_s[pl.ds(g * 16, 16)]
        ov = plsc.load_gather(off_s, [sv])
        dnv = plsc.load_gather(tot_s, [sv])
        nmax = jnp.max(dnv)

        def pair_body(k, accs):
            at, ap = accs
            idx = jnp.minimum(ov + k, EP - 1)
            jv = plsc.load_gather(sidx_s, [idx])
            jv = jnp.where(dnv > k, jv, EP - 1)
            exv = plsc.load_gather(ex_s, [jv])
            eyv = plsc.load_gather(ey_s, [jv])
            ezv = plsc.load_gather(ez_s, [jv])
            d = pxv * exv + pyv * eyv + pzv * ezv
            cx = pyv * ezv - pzv * eyv
            cy = pzv * exv - pxv * ezv
            cz = pxv * eyv - pyv * exv
            c = _sqrt_nn(cx * cx + cy * cy + cz * cz)
            th = _atan2_pos(c, d)
            wv = vxv * exv + vyv * eyv + vzv * ezv
            ph = _atan2_pos(c * jnp.abs(d), wv * d)
            return (at + th, ap + ph)

        z = jnp.zeros((16,), jnp.float32)
        at, ap = lax.fori_loop(0, nmax, pair_body, (z, z))
        t_s[pl.ds(g * 16, 16)] = at
        p_s[pl.ds(g * 16, 16)] = ap
        d_s[pl.ds(g * 16, 16)] = dnv.astype(jnp.float32)
        return 0
    lax.fori_loop(0, EPW // 16, group_body, 0)

    def row_body(g, _):
        tv = t_s[pl.ds(g * 16, 16)]
        pv = p_s[pl.ds(g * 16, 16)]
        dv = d_s[pl.ds(g * 16, 16)]
        for u in range(16):
            row = tv[u] * wth_v + pv[u] * wph_v + dv[u] * ba_v
            az_s[pl.ds((g * 16 + u) * 16, 16)] = row
        return 0
    lax.fori_loop(0, EPW // 16, row_body, 0)

    pltpu.sync_copy(az_s, az_hbm.at[pl.ds(base * 16, EPW * 16)])


def _tc_seg_body(bond_ref, recvc_ref, sendc_ref, ck_ref, cg_ref):
    f32 = jnp.float32
    wb = ck_ref[0:64, :]
    niota_row = lax.broadcasted_iota(jnp.int32, (1, NPAD), 1)

    def segsum(ch, acc):
        sl = pl.ds(ch * 600, 600)
        rc = recvc_ref[sl, :]
        oh = (rc == niota_row).astype(f32)
        bw = jnp.dot(bond_ref[sl, :], wb,
                     preferred_element_type=f32)
        return acc + lax.dot_general(oh, bw, (((0,), (0,)), ((), ())),
                                     preferred_element_type=f32)
    seg = lax.fori_loop(0, 20, segsum, jnp.zeros((NPAD, 16), f32))

    def outt(t, _):
        sl = pl.ds(t * 1000, 1000)
        sc = sendc_ref[sl, :]
        oh = (sc == niota_row).astype(f32)
        cg_ref[sl, :] = jnp.dot(oh, seg, preferred_element_type=f32)
        return 0
    lax.fori_loop(0, 12, outt, 0)


def _tc_add_body(cg_ref, az_ref, cb_ref, out_ref):
    out_ref[...] = cg_ref[...] + az_ref[0:12000, :] + cb_ref[...]


@jax.jit
def kernel(bond_features, local_env, pair_indices, az_kernel, az_bias, concat_kernel, concat_bias):
    E = bond_features.shape[0]
    pad = EP - E
    recv = jnp.pad(pair_indices[:, 1], (0, pad), constant_values=N_NODES)
    send = jnp.pad(pair_indices[:, 0], (0, pad), constant_values=N_NODES + 1)
    exa = jnp.pad(local_env[:, 0], (0, pad))
    eya = jnp.pad(local_env[:, 1], (0, pad))
    eza = jnp.pad(local_env[:, 2], (0, pad))
    vxa = jnp.pad(local_env[:, 3], (0, pad))
    vya = jnp.pad(local_env[:, 4], (0, pad))
    vza = jnp.pad(local_env[:, 5], (0, pad))

    sc_fn = pl.kernel(
        _sc_body,
        out_type=jax.ShapeDtypeStruct((EP * 16,), jnp.float32),
        mesh=plsc.VectorSubcoreMesh(core_axis_name="c", subcore_axis_name="s",
                                    num_cores=2, num_subcores=16),
        compiler_params=pltpu.CompilerParams(needs_layout_passes=False),
        scratch_types=[
            pltpu.VMEM((EP,), jnp.int32),
            pltpu.VMEM((EP,), jnp.float32),
            pltpu.VMEM((EP,), jnp.float32),
            pltpu.VMEM((EP,), jnp.float32),
            pltpu.VMEM((EP,), jnp.int32),
            pltpu.VMEM((16 * NPAD,), jnp.int32),
            pltpu.VMEM((NPAD,), jnp.int32),
            pltpu.VMEM((NPAD,), jnp.int32),
            pltpu.VMEM((EPW,), jnp.int32),
            pltpu.VMEM((EPW,), jnp.float32),
            pltpu.VMEM((EPW,), jnp.float32),
            pltpu.VMEM((EPW,), jnp.float32),
            pltpu.VMEM((EPW,), jnp.float32),
            pltpu.VMEM((EPW,), jnp.float32),
            pltpu.VMEM((EPW,), jnp.float32),
            pltpu.VMEM((EPW * 16,), jnp.float32),
            pltpu.VMEM((128,), jnp.float32),
            pltpu.VMEM((64,), jnp.float32),
            pltpu.VMEM((2048,), jnp.float32),
        ],
    )
    az = sc_fn(recv, send, exa, eya, eza, vxa, vya, vza,
               az_kernel.reshape(-1), az_bias, concat_kernel.reshape(-1))

    cg = pl.pallas_call(
        _tc_seg_body,
        out_shape=jax.ShapeDtypeStruct((E, 16), jnp.float32),
    )(bond_features, pair_indices[:, 1:2], pair_indices[:, 0:1], concat_kernel)

    out = pl.pallas_call(
        _tc_add_body,
        out_shape=jax.ShapeDtypeStruct((E, 16), jnp.float32),
    )(cg, az.reshape(EP, 16), concat_bias[None, :])
    return out


# speedup vs baseline: 1.0404x; 1.0404x over previous
import jax
import jax.numpy as jnp
from jax import lax
from jax.experimental import pallas as pl
from jax.experimental.pallas import tpu as pltpu
from jax.experimental.pallas import tpu_sc as plsc

N_NODES = 1000
EP = 12288
NW = 32
EPW = EP // NW
NPAD = 1024
LPL = EP // 16

_PI = 3.141592653589793
_HALF_PI = 1.5707963267948966


def _atan_poly(a):
    z = a * a
    p = jnp.float32(-0.0117212)
    p = p * z + jnp.float32(0.05265332)
    p = p * z + jnp.float32(-0.11643287)
    p = p * z + jnp.float32(0.19354346)
    p = p * z + jnp.float32(-0.33262347)
    p = p * z + jnp.float32(0.99997726)
    return a * p


def _atan2_pos(y, x):
    ax = jnp.abs(x)
    mn = jnp.minimum(y, ax)
    mx = jnp.maximum(y, ax)
    a = jnp.where(mx > 0.0, mn / mx, 0.0)
    r = _atan_poly(a)
    r = jnp.where(y > ax, _HALF_PI - r, r)
    r = jnp.where(x < 0.0, _PI - r, r)
    return r


def _sqrt_nn(x):
    i = plsc.bitcast(x, jnp.int32)
    i = jnp.int32(0x5F3759DF) - lax.shift_right_logical(i, 1)
    y = plsc.bitcast(i, jnp.float32)
    for _ in range(3):
        y = y * (jnp.float32(1.5) - jnp.float32(0.5) * x * y * y)
    return jnp.where(x > 0.0, x * y, 0.0)


def _sc_body(recv_hbm, send_hbm, ex_hbm, ey_hbm, ez_hbm, vx_hbm, vy_hbm, vz_hbm,
             azk_hbm, azb_hbm, ck_hbm,
             az_hbm,
             recv_s, ex_s, ey_s, ez_s, sidx_s, cnt_s, tot_s, off_s,
             send_s, vx_s, vy_s, vz_s, t_s, p_s, d_s, az_s,
             azk_s, azb_s, ck_s, dsem):
    cid = lax.axis_index("c")
    sid = lax.axis_index("s")
    wid = sid * 2 + cid
    base = wid * EPW

    lanes = jnp.arange(16, dtype=jnp.int32)
    ones_i = jnp.ones((16,), jnp.int32)

    cps = [
        pltpu.async_copy(azk_hbm, azk_s, dsem),
        pltpu.async_copy(azb_hbm, azb_s, dsem),
        pltpu.async_copy(ck_hbm, ck_s, dsem),
        pltpu.async_copy(recv_hbm, recv_s, dsem),
        pltpu.async_copy(ex_hbm, ex_s, dsem),
        pltpu.async_copy(ey_hbm, ey_s, dsem),
        pltpu.async_copy(ez_hbm, ez_s, dsem),
        pltpu.async_copy(send_hbm.at[pl.ds(base, EPW)], send_s, dsem),
        pltpu.async_copy(vx_hbm.at[pl.ds(base, EPW)], vx_s, dsem),
        pltpu.async_copy(vy_hbm.at[pl.ds(base, EPW)], vy_s, dsem),
        pltpu.async_copy(vz_hbm.at[pl.ds(base, EPW)], vz_s, dsem),
    ]
    for cp in cps:
        cp.wait()

    def wgt_body(m, carry):
        wph, wth, ba = carry
        ak0 = azk_s[pl.ds(m * 16, 16)]
        ak1 = azk_s[pl.ds(64 + m * 16, 16)]
        ab = azb_s[pl.ds(m * 16, 16)]
        for u in range(16):
            wa_k = ck_s[pl.ds((64 + m * 16 + u) * 16, 16)]
            wph = wph + ak0[u] * wa_k
            wth = wth + ak1[u] * wa_k
            ba = ba + ab[u] * wa_k
        return (wph, wth, ba)
    zf = jnp.zeros((16,), jnp.float32)
    wph_v, wth_v, ba_v = lax.fori_loop(0, 4, wgt_body, (zf, zf, zf))

    def zero_body(i, _):
        z = jnp.zeros((16,), jnp.int32)
        for u in range(4):
            cnt_s[pl.ds(i * 64 + u * 16, 16)] = z
        return 0
    lax.fori_loop(0, (16 * NPAD) // 64, zero_body, 0)

    def count_body(k, _):
        for u in range(2):
            eidx = lanes * LPL + (k * 2 + u)
            rv = plsc.load_gather(recv_s, [eidx])
            plsc.addupdate_scatter(cnt_s, [lanes * NPAD + rv], ones_i)
        return 0
    lax.fori_loop(0, LPL // 2, count_body, 0)

    def pfx_outer(m, _):
        run = jnp.zeros((16,), jnp.int32)
        for l in range(16):
            o = l * NPAD + m * 16
            cl = cnt_s[pl.ds(o, 16)]
            cnt_s[pl.ds(o, 16)] = run
            run = run + cl
        tot_s[pl.ds(m * 16, 16)] = run
        return 0
    lax.fori_loop(0, NPAD // 16, pfx_outer, 0)

    def off_body(m, carry):
        t = tot_s[pl.ds(m * 16, 16)]
        cs = plsc.cumsum(t)
        off_s[pl.ds(m * 16, 16)] = cs - t + carry
        return carry + jnp.sum(t)
    lax.fori_loop(0, NPAD // 16, off_body, jnp.int32(0))

    def place_body(k, _):
        for u in range(2):
            eidx = lanes * LPL + (k * 2 + u)
            rv = plsc.load_gather(recv_s, [eidx])
            cidx = lanes * NPAD + rv
            pos = plsc.load_gather(off_s, [rv]) + plsc.load_gather(cnt_s, [cidx])
            plsc.store_scatter(sidx_s, [pos], eidx)
            plsc.addupdate_scatter(cnt_s, [cidx], ones_i)
        return 0
    lax.fori_loop(0, LPL // 2, place_body, 0)

    def group_body(g, _):
        gbase = base + g * 16
        sv = send_s[pl.ds(g * 16, 16)]
        pxv = ex_s[pl.ds(gbase, 16)]
        pyv = ey_s[pl.ds(gbase, 16)]
        pzv = ez_s[pl.ds(gbase, 16)]
        vxv = vx_s[pl.ds(g * 16, 16)]
        vyv = vy_s[pl.ds(g * 16, 16)]
        vzv = vz_s[pl.ds(g * 16, 16)]
        ov = plsc.load_gather(off_s, [sv])
        dnv = plsc.load_gather(tot_s, [sv])
        nmax = jnp.max(dnv)

        def pair_body(k, accs):
            at, ap = accs
            idx = jnp.minimum(ov + k, EP - 1)
            jv = plsc.load_gather(sidx_s, [idx])
            jv = jnp.where(dnv > k, jv, EP - 1)
            exv = plsc.load_gather(ex_s, [jv])
            eyv = plsc.load_gather(ey_s, [jv])
            ezv = plsc.load_gather(ez_s, [jv])
            d = pxv * exv + pyv * eyv + pzv * ezv
            cx = pyv * ezv - pzv * eyv
            cy = pzv * exv - pxv * ezv
            cz = pxv * eyv - pyv * exv
            c = _sqrt_nn(cx * cx + cy * cy + cz * cz)
            th = _atan2_pos(c, d)
            wv = vxv * exv + vyv * eyv + vzv * ezv
            ph = _atan2_pos(c * jnp.abs(d), wv * d)
            return (at + th, ap + ph)

        z = jnp.zeros((16,), jnp.float32)
        at, ap = lax.fori_loop(0, nmax, pair_body, (z, z))
        t_s[pl.ds(g * 16, 16)] = at
        p_s[pl.ds(g * 16, 16)] = ap
        d_s[pl.ds(g * 16, 16)] = dnv.astype(jnp.float32)
        return 0
    lax.fori_loop(0, EPW // 16, group_body, 0)

    def row_body(g, _):
        tv = t_s[pl.ds(g * 16, 16)]
        pv = p_s[pl.ds(g * 16, 16)]
        dv = d_s[pl.ds(g * 16, 16)]
        for u in range(16):
            row = tv[u] * wth_v + pv[u] * wph_v + dv[u] * ba_v
            az_s[pl.ds((g * 16 + u) * 16, 16)] = row
        return 0
    lax.fori_loop(0, EPW // 16, row_body, 0)

    pltpu.sync_copy(az_s, az_hbm.at[pl.ds(base * 16, EPW * 16)])


def _tc_seg_body(bond_ref, recvc_ref, sendc_ref, ck_ref, cg_ref):
    f32 = jnp.float32
    wb = ck_ref[0:64, :]
    niota_row = lax.broadcasted_iota(jnp.int32, (1, NPAD), 1)

    def segsum(ch, acc):
        sl = pl.ds(ch * 600, 600)
        rc = recvc_ref[sl, :]
        oh = (rc == niota_row).astype(f32)
        bw = jnp.dot(bond_ref[sl, :], wb,
                     preferred_element_type=f32)
        return acc + lax.dot_general(oh, bw, (((0,), (0,)), ((), ())),
                                     preferred_element_type=f32)
    seg = lax.fori_loop(0, 20, segsum, jnp.zeros((NPAD, 16), f32))

    def outt(t, _):
        sl = pl.ds(t * 1000, 1000)
        sc = sendc_ref[sl, :]
        oh = (sc == niota_row).astype(f32)
        cg_ref[sl, :] = jnp.dot(oh, seg, preferred_element_type=f32)
        return 0
    lax.fori_loop(0, 12, outt, 0)


def _tc_add_body(cg_ref, az_ref, cb_ref, out_ref):
    out_ref[...] = cg_ref[...] + az_ref[0:12000, :] + cb_ref[...]


@jax.jit
def kernel(bond_features, local_env, pair_indices, az_kernel, az_bias, concat_kernel, concat_bias):
    E = bond_features.shape[0]
    pad = EP - E
    recv = jnp.pad(pair_indices[:, 1], (0, pad), constant_values=N_NODES)
    send = jnp.pad(pair_indices[:, 0], (0, pad), constant_values=N_NODES + 1)
    exa = jnp.pad(local_env[:, 0], (0, pad))
    eya = jnp.pad(local_env[:, 1], (0, pad))
    eza = jnp.pad(local_env[:, 2], (0, pad))
    vxa = jnp.pad(local_env[:, 3], (0, pad))
    vya = jnp.pad(local_env[:, 4], (0, pad))
    vza = jnp.pad(local_env[:, 5], (0, pad))

    sc_fn = pl.kernel(
        _sc_body,
        out_type=jax.ShapeDtypeStruct((EP * 16,), jnp.float32),
        mesh=plsc.VectorSubcoreMesh(core_axis_name="c", subcore_axis_name="s",
                                    num_cores=2, num_subcores=16),
        compiler_params=pltpu.CompilerParams(needs_layout_passes=False),
        scratch_types=[
            pltpu.VMEM((EP,), jnp.int32),
            pltpu.VMEM((EP,), jnp.float32),
            pltpu.VMEM((EP,), jnp.float32),
            pltpu.VMEM((EP,), jnp.float32),
            pltpu.VMEM((EP,), jnp.int32),
            pltpu.VMEM((16 * NPAD,), jnp.int32),
            pltpu.VMEM((NPAD,), jnp.int32),
            pltpu.VMEM((NPAD,), jnp.int32),
            pltpu.VMEM((EPW,), jnp.int32),
            pltpu.VMEM((EPW,), jnp.float32),
            pltpu.VMEM((EPW,), jnp.float32),
            pltpu.VMEM((EPW,), jnp.float32),
            pltpu.VMEM((EPW,), jnp.float32),
            pltpu.VMEM((EPW,), jnp.float32),
            pltpu.VMEM((EPW,), jnp.float32),
            pltpu.VMEM((EPW * 16,), jnp.float32),
            pltpu.VMEM((128,), jnp.float32),
            pltpu.VMEM((64,), jnp.float32),
            pltpu.VMEM((2048,), jnp.float32),
            pltpu.SemaphoreType.DMA,
        ],
    )
    az = sc_fn(recv, send, exa, eya, eza, vxa, vya, vza,
               az_kernel.reshape(-1), az_bias, concat_kernel.reshape(-1))

    cg = pl.pallas_call(
        _tc_seg_body,
        out_shape=jax.ShapeDtypeStruct((E, 16), jnp.float32),
    )(bond_features, pair_indices[:, 1:2], pair_indices[:, 0:1], concat_kernel)

    out = pl.pallas_call(
        _tc_add_body,
        out_shape=jax.ShapeDtypeStruct((E, 16), jnp.float32),
    )(cg, az.reshape(EP, 16), concat_bias[None, :])
    return out


# speedup vs baseline: 1.0852x; 1.0430x over previous
import jax
import jax.numpy as jnp
from jax import lax
from jax.experimental import pallas as pl
from jax.experimental.pallas import tpu as pltpu
from jax.experimental.pallas import tpu_sc as plsc

N_NODES = 1000
EP = 12288
NW = 32
EPW = EP // NW
NPAD = 1024
LPL = EP // 16

_PI = 3.141592653589793
_HALF_PI = 1.5707963267948966


def _atan_poly(a):
    z = a * a
    p = jnp.float32(-0.0117212)
    p = p * z + jnp.float32(0.05265332)
    p = p * z + jnp.float32(-0.11643287)
    p = p * z + jnp.float32(0.19354346)
    p = p * z + jnp.float32(-0.33262347)
    p = p * z + jnp.float32(0.99997726)
    return a * p


def _atan2_pos(y, x):
    ax = jnp.abs(x)
    mn = jnp.minimum(y, ax)
    mx = jnp.maximum(y, ax)
    a = jnp.where(mx > 0.0, mn / mx, 0.0)
    r = _atan_poly(a)
    r = jnp.where(y > ax, _HALF_PI - r, r)
    r = jnp.where(x < 0.0, _PI - r, r)
    return r


def _sqrt_nn(x):
    i = plsc.bitcast(x, jnp.int32)
    i = jnp.int32(0x5F3759DF) - lax.shift_right_logical(i, 1)
    y = plsc.bitcast(i, jnp.float32)
    for _ in range(3):
        y = y * (jnp.float32(1.5) - jnp.float32(0.5) * x * y * y)
    return jnp.where(x > 0.0, x * y, 0.0)


def _sc_body(recv_hbm, send_hbm, ex_hbm, ey_hbm, ez_hbm, vx_hbm, vy_hbm, vz_hbm,
             azk_hbm, azb_hbm, ck_hbm,
             az_hbm,
             recv_s, ex_s, ey_s, ez_s, sidx_s, cnt_s, tot_s, off_s,
             send_s, vx_s, vy_s, vz_s, t_s, p_s, d_s, az_s,
             azk_s, azb_s, ck_s, dsem):
    cid = lax.axis_index("c")
    sid = lax.axis_index("s")
    wid = sid * 2 + cid
    base = wid * EPW

    lanes = jnp.arange(16, dtype=jnp.int32)
    ones_i = jnp.ones((16,), jnp.int32)

    cps = [
        pltpu.async_copy(azk_hbm, azk_s, dsem),
        pltpu.async_copy(azb_hbm, azb_s, dsem),
        pltpu.async_copy(ck_hbm, ck_s, dsem),
        pltpu.async_copy(recv_hbm, recv_s, dsem),
        pltpu.async_copy(ex_hbm, ex_s, dsem),
        pltpu.async_copy(ey_hbm, ey_s, dsem),
        pltpu.async_copy(ez_hbm, ez_s, dsem),
        pltpu.async_copy(send_hbm.at[pl.ds(base, EPW)], send_s, dsem),
        pltpu.async_copy(vx_hbm.at[pl.ds(base, EPW)], vx_s, dsem),
        pltpu.async_copy(vy_hbm.at[pl.ds(base, EPW)], vy_s, dsem),
        pltpu.async_copy(vz_hbm.at[pl.ds(base, EPW)], vz_s, dsem),
    ]
    for cp in cps:
        cp.wait()

    def wgt_body(m, carry):
        wph, wth, ba = carry
        ak0 = azk_s[pl.ds(m * 16, 16)]
        ak1 = azk_s[pl.ds(64 + m * 16, 16)]
        ab = azb_s[pl.ds(m * 16, 16)]
        for u in range(16):
            wa_k = ck_s[pl.ds((64 + m * 16 + u) * 16, 16)]
            wph = wph + ak0[u] * wa_k
            wth = wth + ak1[u] * wa_k
            ba = ba + ab[u] * wa_k
        return (wph, wth, ba)
    zf = jnp.zeros((16,), jnp.float32)
    wph_v, wth_v, ba_v = lax.fori_loop(0, 4, wgt_body, (zf, zf, zf))

    def zero_body(i, _):
        z = jnp.zeros((16,), jnp.int32)
        for u in range(8):
            cnt_s[pl.ds(i * 128 + u * 16, 16)] = z
        return 0
    lax.fori_loop(0, (16 * NPAD) // 128, zero_body, 0)

    def count_body(k, _):
        for u in range(6):
            eidx = lanes * LPL + (k * 6 + u)
            rv = plsc.load_gather(recv_s, [eidx])
            plsc.addupdate_scatter(cnt_s, [lanes * NPAD + rv], ones_i)
        return 0
    lax.fori_loop(0, LPL // 6, count_body, 0)

    def pfx_outer(m, _):
        run = jnp.zeros((16,), jnp.int32)
        for l in range(16):
            o = l * NPAD + m * 16
            cl = cnt_s[pl.ds(o, 16)]
            cnt_s[pl.ds(o, 16)] = run
            run = run + cl
        tot_s[pl.ds(m * 16, 16)] = run
        return 0
    lax.fori_loop(0, NPAD // 16, pfx_outer, 0)

    def off_body(m, carry):
        t = tot_s[pl.ds(m * 16, 16)]
        cs = plsc.cumsum(t)
        off_s[pl.ds(m * 16, 16)] = cs - t + carry
        return carry + jnp.sum(t)
    lax.fori_loop(0, NPAD // 16, off_body, jnp.int32(0))

    def place_body(k, _):
        for u in range(6):
            eidx = lanes * LPL + (k * 6 + u)
            rv = plsc.load_gather(recv_s, [eidx])
            cidx = lanes * NPAD + rv
            pos = plsc.load_gather(off_s, [rv]) + plsc.load_gather(cnt_s, [cidx])
            plsc.store_scatter(sidx_s, [pos], eidx)
            plsc.addupdate_scatter(cnt_s, [cidx], ones_i)
        return 0
    lax.fori_loop(0, LPL // 6, place_body, 0)

    def group_body(g, _):
        gbase = base + g * 16
        sv = send_s[pl.ds(g * 16, 16)]
        pxv = ex_s[pl.ds(gbase, 16)]
        pyv = ey_s[pl.ds(gbase, 16)]
        pzv = ez_s[pl.ds(gbase, 16)]
        vxv = vx_s[pl.ds(g * 16, 16)]
        vyv = vy_s[pl.ds(g * 16, 16)]
        vzv = vz_s[pl.ds(g * 16, 16)]
        ov = plsc.load_gather(off_s, [sv])
        dnv = plsc.load_gather(tot_s, [sv])
        nmax = jnp.max(dnv)

        def pair_body(k, accs):
            at, ap = accs
            idx = jnp.minimum(ov + k, EP - 1)
            jv = plsc.load_gather(sidx_s, [idx])
            jv = jnp.where(dnv > k, jv, EP - 1)
            exv = plsc.load_gather(ex_s, [jv])
            eyv = plsc.load_gather(ey_s, [jv])
            ezv = plsc.load_gather(ez_s, [jv])
            d = pxv * exv + pyv * eyv + pzv * ezv
            cx = pyv * ezv - pzv * eyv
            cy = pzv * exv - pxv * ezv
            cz = pxv * eyv - pyv * exv
            c = _sqrt_nn(cx * cx + cy * cy + cz * cz)
            th = _atan2_pos(c, d)
            wv = vxv * exv + vyv * eyv + vzv * ezv
            ph = _atan2_pos(c * jnp.abs(d), wv * d)
            return (at + th, ap + ph)

        z = jnp.zeros((16,), jnp.float32)
        at, ap = lax.fori_loop(0, nmax, pair_body, (z, z))
        t_s[pl.ds(g * 16, 16)] = at
        p_s[pl.ds(g * 16, 16)] = ap
        d_s[pl.ds(g * 16, 16)] = dnv.astype(jnp.float32)
        return 0
    lax.fori_loop(0, EPW // 16, group_body, 0)

    def row_body(g, _):
        tv = t_s[pl.ds(g * 16, 16)]
        pv = p_s[pl.ds(g * 16, 16)]
        dv = d_s[pl.ds(g * 16, 16)]
        for u in range(16):
            row = tv[u] * wth_v + pv[u] * wph_v + dv[u] * ba_v
            az_s[pl.ds((g * 16 + u) * 16, 16)] = row
        return 0
    lax.fori_loop(0, EPW // 16, row_body, 0)

    pltpu.sync_copy(az_s, az_hbm.at[pl.ds(base * 16, EPW * 16)])


def _tc_seg_body(bond_ref, recvc_ref, sendc_ref, ck_ref, cg_ref):
    f32 = jnp.float32
    wb = ck_ref[0:64, :]
    niota_row = lax.broadcasted_iota(jnp.int32, (1, NPAD), 1)

    def segsum(ch, acc):
        sl = pl.ds(ch * 600, 600)
        rc = recvc_ref[sl, :]
        oh = (rc == niota_row).astype(f32)
        bw = jnp.dot(bond_ref[sl, :], wb,
                     preferred_element_type=f32)
        return acc + lax.dot_general(oh, bw, (((0,), (0,)), ((), ())),
                                     preferred_element_type=f32)
    seg = lax.fori_loop(0, 20, segsum, jnp.zeros((NPAD, 16), f32))

    def outt(t, _):
        sl = pl.ds(t * 1000, 1000)
        sc = sendc_ref[sl, :]
        oh = (sc == niota_row).astype(f32)
        cg_ref[sl, :] = jnp.dot(oh, seg, preferred_element_type=f32)
        return 0
    lax.fori_loop(0, 12, outt, 0)


def _tc_add_body(cg_ref, az_ref, cb_ref, out_ref):
    out_ref[...] = cg_ref[...] + az_ref[0:12000, :] + cb_ref[...]


@jax.jit
def kernel(bond_features, local_env, pair_indices, az_kernel, az_bias, concat_kernel, concat_bias):
    E = bond_features.shape[0]
    pad = EP - E
    recv = jnp.pad(pair_indices[:, 1], (0, pad), constant_values=N_NODES)
    send = jnp.pad(pair_indices[:, 0], (0, pad), constant_values=N_NODES + 1)
    exa = jnp.pad(local_env[:, 0], (0, pad))
    eya = jnp.pad(local_env[:, 1], (0, pad))
    eza = jnp.pad(local_env[:, 2], (0, pad))
    vxa = jnp.pad(local_env[:, 3], (0, pad))
    vya = jnp.pad(local_env[:, 4], (0, pad))
    vza = jnp.pad(local_env[:, 5], (0, pad))

    sc_fn = pl.kernel(
        _sc_body,
        out_type=jax.ShapeDtypeStruct((EP * 16,), jnp.float32),
        mesh=plsc.VectorSubcoreMesh(core_axis_name="c", subcore_axis_name="s",
                                    num_cores=2, num_subcores=16),
        compiler_params=pltpu.CompilerParams(needs_layout_passes=False),
        scratch_types=[
            pltpu.VMEM((EP,), jnp.int32),
            pltpu.VMEM((EP,), jnp.float32),
            pltpu.VMEM((EP,), jnp.float32),
            pltpu.VMEM((EP,), jnp.float32),
            pltpu.VMEM((EP,), jnp.int32),
            pltpu.VMEM((16 * NPAD,), jnp.int32),
            pltpu.VMEM((NPAD,), jnp.int32),
            pltpu.VMEM((NPAD,), jnp.int32),
            pltpu.VMEM((EPW,), jnp.int32),
            pltpu.VMEM((EPW,), jnp.float32),
            pltpu.VMEM((EPW,), jnp.float32),
            pltpu.VMEM((EPW,), jnp.float32),
            pltpu.VMEM((EPW,), jnp.float32),
            pltpu.VMEM((EPW,), jnp.float32),
            pltpu.VMEM((EPW,), jnp.float32),
            pltpu.VMEM((EPW * 16,), jnp.float32),
            pltpu.VMEM((128,), jnp.float32),
            pltpu.VMEM((64,), jnp.float32),
            pltpu.VMEM((2048,), jnp.float32),
            pltpu.SemaphoreType.DMA,
        ],
    )
    az = sc_fn(recv, send, exa, eya, eza, vxa, vya, vza,
               az_kernel.reshape(-1), az_bias, concat_kernel.reshape(-1))

    cg = pl.pallas_call(
        _tc_seg_body,
        out_shape=jax.ShapeDtypeStruct((E, 16), jnp.float32),
    )(bond_features, pair_indices[:, 1:2], pair_indices[:, 0:1], concat_kernel)

    out = pl.pallas_call(
        _tc_add_body,
        out_shape=jax.ShapeDtypeStruct((E, 16), jnp.float32),
    )(cg, az.reshape(EP, 16), concat_bias[None, :])
    return out
